# baseline (speedup 1.0000x reference)
import functools

import jax
import jax.numpy as jnp
from jax import lax
from jax.experimental import pallas as pl
from jax.experimental.pallas import tpu as pltpu
from jax.experimental.pallas import tpu_sc as plsc

NUM_WORKERS = 32
SHORT = 32
DIM = 128


WORDS = SHORT // 2


def _sc_gather(ids_flat, table, n_rows):
    b_per_w = n_rows // NUM_WORKERS
    chunk = b_per_w // 2
    mesh = plsc.VectorSubcoreMesh(core_axis_name="c", subcore_axis_name="s")

    @functools.partial(
        pl.kernel,
        mesh=mesh,
        out_type=jax.ShapeDtypeStruct((n_rows, SHORT), jnp.float32),
        scratch_types=[
            pltpu.VMEM((chunk,), jnp.int32),
            pltpu.VMEM((chunk, SHORT), jnp.float32),
            pltpu.SemaphoreType.DMA,
        ],
        compiler_params=pltpu.CompilerParams(use_tc_tiling_on_sc=False),
    )
    def gather_kernel(ids_hbm, table_hbm, out_hbm, idx_v, rows_v, sem):
        wid = lax.axis_index("s") * 2 + lax.axis_index("c")
        for j in range(2):
            base = wid * b_per_w + j * chunk
            pltpu.sync_copy(ids_hbm.at[pl.ds(base, chunk)], idx_v)
            pltpu.async_copy(table_hbm.at[idx_v], rows_v, sem).wait()
            pltpu.sync_copy(rows_v, out_hbm.at[pl.ds(base, chunk)])

    return gather_kernel(ids_flat, table)


def _repack_body(et_ref, eye_ref, o_ref):
    o_ref[...] = jax.lax.dot_general(
        et_ref[...],
        eye_ref[...],
        (((0,), (0,)), ((), ())),
        preferred_element_type=jnp.float32,
    )


def _tc_wordview(eT, num_emb):
    c = 16384
    eye = jnp.eye(SHORT, dtype=jnp.bfloat16)
    return pl.pallas_call(
        _repack_body,
        grid=(pl.cdiv(num_emb, c),),
        in_specs=[
            pl.BlockSpec((SHORT, c), lambda i: (0, i)),
            pl.BlockSpec((SHORT, SHORT), lambda i: (0, 0)),
        ],
        out_specs=pl.BlockSpec((c, SHORT), lambda i: (i, 0)),
        out_shape=jax.ShapeDtypeStruct((num_emb, SHORT), jnp.float32),
    )(eT, eye)


def _proj_body(x_ref, w_ref, b_ref, o_ref):
    acc = jnp.dot(x_ref[...], w_ref[...], preferred_element_type=jnp.float32)
    o_ref[...] = (acc + b_ref[...]).astype(jnp.bfloat16)


def _tc_project(xw, wbd, b8, n_lines):
    block = 3200
    return pl.pallas_call(
        _proj_body,
        grid=(n_lines // block,),
        in_specs=[
            pl.BlockSpec((block, 8 * SHORT), lambda i: (i, 0)),
            pl.BlockSpec((8 * SHORT, 8 * DIM), lambda i: (0, 0)),
            pl.BlockSpec((1, 8 * DIM), lambda i: (0, 0)),
        ],
        out_specs=pl.BlockSpec((block, 8 * DIM), lambda i: (i, 0)),
        out_shape=jax.ShapeDtypeStruct((n_lines, 8 * DIM), jnp.bfloat16),
    )(xw, wbd, b8)


def kernel(ids, embed, W, b):
    B, L = ids.shape
    n_rows = B * L
    num_emb = embed.shape[0]
    ids_flat = ids.T.reshape(n_rows).astype(jnp.int32)
    table_f32 = _tc_wordview(embed.T, num_emb)
    x2 = _sc_gather(ids_flat, table_f32, n_rows)
    n_lines = n_rows // 8
    xw = x2.reshape(n_lines, 8 * SHORT)
    wc = W.astype(jnp.bfloat16).astype(jnp.float32)
    wbd = jnp.kron(jnp.eye(8, dtype=jnp.float32), wc.T)
    b8 = jnp.tile(
        b.astype(jnp.bfloat16).astype(jnp.float32), 8
    ).reshape(1, 8 * DIM)
    out8 = _tc_project(xw, wbd, b8, n_lines)
    out = out8.reshape(n_rows, DIM)
    return out.reshape(L, B, DIM).transpose(1, 0, 2)


# speedup vs baseline: 1.0818x; 1.0818x over previous
import functools

import jax
import jax.numpy as jnp
from jax import lax
from jax.experimental import pallas as pl
from jax.experimental.pallas import tpu as pltpu
from jax.experimental.pallas import tpu_sc as plsc

NUM_WORKERS = 32
SHORT = 32
DIM = 128
WORDS = SHORT // 2


def _sc_gather(ids_flat, table, n_rows):
    b_per_w = n_rows // NUM_WORKERS
    mesh = plsc.VectorSubcoreMesh(core_axis_name="c", subcore_axis_name="s")

    @functools.partial(
        pl.kernel,
        mesh=mesh,
        out_type=jax.ShapeDtypeStruct((n_rows, WORDS), jnp.int32),
        scratch_types=[
            pltpu.VMEM((b_per_w,), jnp.int32),
            pltpu.VMEM((b_per_w, WORDS), jnp.int32),
            pltpu.SemaphoreType.DMA,
        ],
        compiler_params=pltpu.CompilerParams(use_tc_tiling_on_sc=False),
    )
    def gather_kernel(ids_hbm, table_hbm, out_hbm, idx_v, rows_v, sem):
        wid = lax.axis_index("s") * 2 + lax.axis_index("c")
        base = wid * b_per_w
        pltpu.sync_copy(ids_hbm.at[pl.ds(base, b_per_w)], idx_v)
        pltpu.async_copy(table_hbm.at[idx_v], rows_v, sem).wait()
        pltpu.sync_copy(rows_v, out_hbm.at[pl.ds(base, b_per_w)])

    return gather_kernel(ids_flat, table)


def _repack_body(et_ref, o_ref):
    o_ref[...] = et_ref.bitcast(jnp.int32)[...].T


def _tc_wordview(eT, num_emb):
    c = 16384
    return pl.pallas_call(
        _repack_body,
        grid=(pl.cdiv(num_emb, c),),
        in_specs=[pl.BlockSpec((2 * WORDS, c), lambda i: (0, i))],
        out_specs=pl.BlockSpec((c, WORDS), lambda i: (i, 0)),
        out_shape=jax.ShapeDtypeStruct((num_emb, WORDS), jnp.int32),
    )(eT)


def _proj_body(x_ref, we_ref, wo_ref, b_ref, o_ref):
    xw = x_ref[...]
    e = jax.lax.bitcast_convert_type(xw << 16, jnp.float32)
    o = jax.lax.bitcast_convert_type(xw & jnp.int32(-65536), jnp.float32)
    acc = jnp.dot(e, we_ref[...], preferred_element_type=jnp.float32)
    acc += jnp.dot(o, wo_ref[...], preferred_element_type=jnp.float32)
    o_ref[...] = (acc + b_ref[...]).astype(jnp.bfloat16)


def _tc_project(xw, we, wo, b8, n_lines):
    block = 3200
    return pl.pallas_call(
        _proj_body,
        grid=(n_lines // block,),
        in_specs=[
            pl.BlockSpec((block, 8 * WORDS), lambda i: (i, 0)),
            pl.BlockSpec((8 * WORDS, 8 * DIM), lambda i: (0, 0)),
            pl.BlockSpec((8 * WORDS, 8 * DIM), lambda i: (0, 0)),
            pl.BlockSpec((1, 8 * DIM), lambda i: (0, 0)),
        ],
        out_specs=pl.BlockSpec((block, 8 * DIM), lambda i: (i, 0)),
        out_shape=jax.ShapeDtypeStruct((n_lines, 8 * DIM), jnp.bfloat16),
    )(xw, we, wo, b8)


def kernel(ids, embed, W, b):
    B, L = ids.shape
    n_rows = B * L
    num_emb = embed.shape[0]
    ids_flat = ids.T.reshape(n_rows).astype(jnp.int32)
    table_i32 = _tc_wordview(embed.T, num_emb)
    x2 = _sc_gather(ids_flat, table_i32, n_rows)
    n_lines = n_rows // 8
    xw = x2.reshape(n_lines, 8 * WORDS)
    wc = W.astype(jnp.bfloat16).astype(jnp.float32)
    we = jnp.kron(jnp.eye(8, dtype=jnp.float32), wc[:, 0::2].T)
    wo = jnp.kron(jnp.eye(8, dtype=jnp.float32), wc[:, 1::2].T)
    b8 = jnp.tile(
        b.astype(jnp.bfloat16).astype(jnp.float32), 8
    ).reshape(1, 8 * DIM)
    out8 = _tc_project(xw, we, wo, b8, n_lines)
    out = out8.reshape(n_rows, DIM)
    return out.reshape(L, B, DIM).transpose(1, 0, 2)


# speedup vs baseline: 1.0896x; 1.0072x over previous
import functools

import jax
import jax.numpy as jnp
from jax import lax
from jax.experimental import pallas as pl
from jax.experimental.pallas import tpu as pltpu
from jax.experimental.pallas import tpu_sc as plsc

NUM_WORKERS = 32
SHORT = 32
DIM = 128
WORDS = SHORT // 2


def _sc_gather(ids_flat, table, n_rows):
    b_per_w = n_rows // NUM_WORKERS
    mesh = plsc.VectorSubcoreMesh(core_axis_name="c", subcore_axis_name="s")

    @functools.partial(
        pl.kernel,
        mesh=mesh,
        out_type=jax.ShapeDtypeStruct((n_rows, WORDS), jnp.int32),
        scratch_types=[
            pltpu.VMEM((b_per_w,), jnp.int32),
            pltpu.VMEM((b_per_w, WORDS), jnp.int32),
            pltpu.SemaphoreType.DMA,
        ],
        compiler_params=pltpu.CompilerParams(use_tc_tiling_on_sc=False),
    )
    def gather_kernel(ids_hbm, table_hbm, out_hbm, idx_v, rows_v, sem):
        wid = lax.axis_index("s") * 2 + lax.axis_index("c")
        base = wid * b_per_w
        pltpu.sync_copy(ids_hbm.at[pl.ds(base, b_per_w)], idx_v)
        pltpu.async_copy(table_hbm.at[idx_v], rows_v, sem).wait()
        pltpu.sync_copy(rows_v, out_hbm.at[pl.ds(base, b_per_w)])

    return gather_kernel(ids_flat, table)


def _repack_body(et_ref, o_ref):
    o_ref[...] = et_ref.bitcast(jnp.int32)[...].T


def _tc_wordview(eT, num_emb):
    c = 32768
    return pl.pallas_call(
        _repack_body,
        grid=(pl.cdiv(num_emb, c),),
        in_specs=[pl.BlockSpec((2 * WORDS, c), lambda i: (0, i))],
        out_specs=pl.BlockSpec((c, WORDS), lambda i: (i, 0)),
        out_shape=jax.ShapeDtypeStruct((num_emb, WORDS), jnp.int32),
    )(eT)


def _proj_body(x_ref, we_ref, wo_ref, b_ref, o_ref):
    xw = x_ref[...]
    e = jax.lax.bitcast_convert_type(xw << 16, jnp.float32)
    o = jax.lax.bitcast_convert_type(xw & jnp.int32(-65536), jnp.float32)
    acc = jnp.dot(e, we_ref[...], preferred_element_type=jnp.float32)
    acc += jnp.dot(o, wo_ref[...], preferred_element_type=jnp.float32)
    o_ref[...] = (acc + b_ref[...]).astype(jnp.bfloat16)


def _tc_project(xw, we, wo, b8, n_lines):
    block = 3200
    return pl.pallas_call(
        _proj_body,
        grid=(n_lines // block,),
        in_specs=[
            pl.BlockSpec((block, 8 * WORDS), lambda i: (i, 0)),
            pl.BlockSpec((8 * WORDS, 8 * DIM), lambda i: (0, 0)),
            pl.BlockSpec((8 * WORDS, 8 * DIM), lambda i: (0, 0)),
            pl.BlockSpec((1, 8 * DIM), lambda i: (0, 0)),
        ],
        out_specs=pl.BlockSpec((block, 8 * DIM), lambda i: (i, 0)),
        out_shape=jax.ShapeDtypeStruct((n_lines, 8 * DIM), jnp.bfloat16),
    )(xw, we, wo, b8)


def kernel(ids, embed, W, b):
    B, L = ids.shape
    n_rows = B * L
    num_emb = embed.shape[0]
    ids_flat = ids.T.reshape(n_rows).astype(jnp.int32)
    table_i32 = _tc_wordview(embed.T, num_emb)
    x2 = _sc_gather(ids_flat, table_i32, n_rows)
    n_lines = n_rows // 8
    xw = x2.reshape(n_lines, 8 * WORDS)
    wc = W.astype(jnp.bfloat16).astype(jnp.float32)
    we = jnp.kron(jnp.eye(8, dtype=jnp.float32), wc[:, 0::2].T)
    wo = jnp.kron(jnp.eye(8, dtype=jnp.float32), wc[:, 1::2].T)
    b8 = jnp.tile(
        b.astype(jnp.bfloat16).astype(jnp.float32), 8
    ).reshape(1, 8 * DIM)
    out8 = _tc_project(xw, we, wo, b8, n_lines)
    out = out8.reshape(n_rows, DIM)
    return out.reshape(L, B, DIM).transpose(1, 0, 2)
